# Initial kernel scaffold; baseline (speedup 1.0000x reference)
import jax
import jax.numpy as jnp
from jax.experimental import pallas as pl


def kernel(in_feat, edge_index, W0, b0, W1, b1, W2, b2):
    raise NotImplementedError("write your pallas kernel here")


# speedup vs baseline: 4.9664x; 4.9664x over previous
import functools

import jax
import jax.numpy as jnp
from jax import lax
from jax.experimental import pallas as pl
from jax.experimental.pallas import tpu as pltpu
from jax.experimental.pallas import tpu_sc as plsc

N = 10000
D = 256
DH = 128
N_PAD = 10240
ROWS_PER_TILE = 640
E_PAD = 161792
EDGES_PER_TILE = 10112
K = 128
NCHUNK = EDGES_PER_TILE // K
TRASH = 10016

def _deg_body(srcp, dstp, cnt_src, cnt_dst, sbuf, dbuf, tbuf, ones_b,
              obuf, acc):
    c = lax.axis_index("c")
    s = lax.axis_index("s")
    base_r = s * ROWS_PER_TILE
    ebase = s * EDGES_PER_TILE

    one16 = jnp.ones((16,), jnp.float32)
    zero16 = jnp.zeros((16,), jnp.float32)
    for j in range(K):
        ones_b[j] = one16
    for j in range(K):
        obuf[j] = zero16

    def zinit(k, carry):
        pltpu.sync_copy(obuf, acc.at[pl.ds(base_r + k * K, K)])
        return carry
    lax.fori_loop(0, ROWS_PER_TILE // K, zinit, 0)
    plsc.subcore_barrier()

    def run(use_src):
        def chunk(g, carry):
            off = ebase + g * K
            pltpu.sync_copy(srcp.at[pl.ds(off, K)], sbuf)
            pltpu.sync_copy(dstp.at[pl.ds(off, K)], dbuf)
            for j in range(K // 16):
                sv = sbuf[pl.ds(j * 16, 16)]
                dv = dbuf[pl.ds(j * 16, 16)]
                key = sv if use_src else dv
                tbuf[pl.ds(j * 16, 16)] = jnp.where(sv != dv, key, TRASH)
            pltpu.sync_copy(ones_b, acc.at[tbuf], add=True)
            return carry
        lax.fori_loop(0, NCHUNK, chunk, 0)

    @pl.when(c == 0)
    def _():
        run(True)

    @pl.when(c == 1)
    def _():
        run(False)

    plsc.subcore_barrier()

    def wb(k, carry):
        pltpu.sync_copy(acc.at[pl.ds(base_r + k * K, K)], obuf)

        @pl.when(c == 0)
        def _():
            pltpu.sync_copy(obuf, cnt_src.at[pl.ds(base_r + k * K, K)])

        @pl.when(c == 1)
        def _():
            pltpu.sync_copy(obuf, cnt_dst.at[pl.ds(base_r + k * K, K)])
        return carry
    lax.fori_loop(0, ROWS_PER_TILE // K, wb, 0)


def _agg_body(h_lo, h_hi, srcp, dstp, agg_lo, agg_hi, sbuf, dbuf, tbuf,
              rows, ibuf, acc, sem):
    c = lax.axis_index("c")
    s = lax.axis_index("s")
    base_r = s * ROWS_PER_TILE
    ebase = s * EDGES_PER_TILE

    def run(h_ref, out_ref):
        def init_k(k, carry):
            pltpu.sync_copy(h_ref.at[pl.ds(base_r + k * K, K)], ibuf)
            pltpu.sync_copy(ibuf, acc.at[pl.ds(base_r + k * K, K)])
            return carry
        lax.fori_loop(0, ROWS_PER_TILE // K, init_k, 0)
        plsc.subcore_barrier()

        def chunk(g, carry):
            off = ebase + g * K
            pltpu.sync_copy(srcp.at[pl.ds(off, K)], sbuf)
            pltpu.sync_copy(dstp.at[pl.ds(off, K)], dbuf)
            for j in range(K // 16):
                sv = sbuf[pl.ds(j * 16, 16)]
                dv = dbuf[pl.ds(j * 16, 16)]
                tbuf[pl.ds(j * 16, 16)] = jnp.where(sv != dv, dv, TRASH)
            pltpu.async_copy(h_ref.at[sbuf], rows, sem).wait()
            pltpu.sync_copy(rows, acc.at[tbuf], add=True)
            return carry
        lax.fori_loop(0, NCHUNK, chunk, 0)
        plsc.subcore_barrier()

        def wb(k, carry):
            pltpu.sync_copy(acc.at[pl.ds(base_r + k * K, K)], ibuf)
            pltpu.sync_copy(ibuf, out_ref.at[pl.ds(base_r + k * K, K)])
            return carry
        lax.fori_loop(0, ROWS_PER_TILE // K, wb, 0)

    @pl.when(c == 0)
    def _():
        run(h_lo, agg_lo)

    @pl.when(c == 1)
    def _():
        run(h_hi, agg_hi)


_BLK = 1024
_GRID = N_PAD // _BLK


def _mm0_body(x_ref, cs_ref, w_ref, lo_ref, hi_ref):
    scale = lax.rsqrt(cs_ref[:, 0:1] + 1.0)
    h = jnp.dot(x_ref[:] * scale, w_ref[:],
                preferred_element_type=jnp.float32)
    lo_ref[:] = h[:, :DH]
    hi_ref[:] = h[:, DH:]


def _mid_body(lo_ref, hi_ref, cd_ref, cs_ref, b_ref, w_ref, olo_ref,
              ohi_ref):
    agg = jnp.concatenate([lo_ref[:], hi_ref[:]], axis=1)
    iscale = lax.rsqrt(cd_ref[:, 0:1] + 1.0)
    x2 = jnp.maximum(agg * iscale + b_ref[:], 0.0)
    oscale = lax.rsqrt(cs_ref[:, 0:1] + 1.0)
    h = jnp.dot(x2 * oscale, w_ref[:], preferred_element_type=jnp.float32)
    olo_ref[:] = h[:, :DH]
    ohi_ref[:] = h[:, DH:]


def _fin_body(lo_ref, hi_ref, cd_ref, b_ref, out_ref):
    agg = jnp.concatenate([lo_ref[:], hi_ref[:]], axis=1)
    iscale = lax.rsqrt(cd_ref[:, 0:1] + 1.0)
    out_ref[:] = jnp.maximum(agg * iscale + b_ref[:], 0.0)


def _rows_spec(width):
    return pl.BlockSpec((_BLK, width), lambda i: (i, 0))


def _full_spec(shape):
    return pl.BlockSpec(shape, lambda i: (0,) * len(shape))


_mm0 = pl.pallas_call(
    _mm0_body,
    grid=(_GRID,),
    in_specs=[_rows_spec(D), _rows_spec(16), _full_spec((D, D))],
    out_specs=[_rows_spec(DH), _rows_spec(DH)],
    out_shape=[jax.ShapeDtypeStruct((N_PAD, DH), jnp.float32),
               jax.ShapeDtypeStruct((N_PAD, DH), jnp.float32)],
)

_mid = pl.pallas_call(
    _mid_body,
    grid=(_GRID,),
    in_specs=[_rows_spec(DH), _rows_spec(DH), _rows_spec(16),
              _rows_spec(16), _full_spec((1, D)), _full_spec((D, D))],
    out_specs=[_rows_spec(DH), _rows_spec(DH)],
    out_shape=[jax.ShapeDtypeStruct((N_PAD, DH), jnp.float32),
               jax.ShapeDtypeStruct((N_PAD, DH), jnp.float32)],
)

_FBLK = 1000
_fin = pl.pallas_call(
    _fin_body,
    grid=(N // _FBLK,),
    in_specs=[pl.BlockSpec((_FBLK, DH), lambda i: (i, 0)),
              pl.BlockSpec((_FBLK, DH), lambda i: (i, 0)),
              pl.BlockSpec((_FBLK, 16), lambda i: (i, 0)),
              _full_spec((1, D))],
    out_specs=pl.BlockSpec((_FBLK, D), lambda i: (i, 0)),
    out_shape=jax.ShapeDtypeStruct((N, D), jnp.float32),
)


@functools.cache
def _sc_kernels():
    mesh = plsc.VectorSubcoreMesh(core_axis_name="c", subcore_axis_name="s")
    deg = pl.kernel(
        _deg_body,
        mesh=mesh,
        out_type=(
            jax.ShapeDtypeStruct((N_PAD, 16), jnp.float32),
            jax.ShapeDtypeStruct((N_PAD, 16), jnp.float32),
        ),
        scratch_types=[
            pltpu.VMEM((K,), jnp.int32),
            pltpu.VMEM((K,), jnp.int32),
            pltpu.VMEM((K,), jnp.int32),
            pltpu.VMEM((K, 16), jnp.float32),
            pltpu.VMEM((K, 16), jnp.float32),
            pltpu.VMEM_SHARED((N_PAD, 16), jnp.float32),
        ],
    )
    agg = pl.kernel(
        _agg_body,
        mesh=mesh,
        out_type=(
            jax.ShapeDtypeStruct((N_PAD, DH), jnp.float32),
            jax.ShapeDtypeStruct((N_PAD, DH), jnp.float32),
        ),
        scratch_types=[
            pltpu.VMEM((K,), jnp.int32),
            pltpu.VMEM((K,), jnp.int32),
            pltpu.VMEM((K,), jnp.int32),
            pltpu.VMEM((K, DH), jnp.float32),
            pltpu.VMEM((K, DH), jnp.float32),
            pltpu.VMEM_SHARED((N_PAD, DH), jnp.float32),
            pltpu.SemaphoreType.DMA,
        ],
    )
    return deg, agg


def kernel(in_feat, edge_index, W0, b0, W1, b1, W2, b2):
    _deg_kernel, _agg_kernel = _sc_kernels()
    src = edge_index[0]
    dst = edge_index[1]
    pad = jnp.zeros((E_PAD - src.shape[0],), jnp.int32)
    srcp = jnp.concatenate([src, pad])
    dstp = jnp.concatenate([dst, pad])
    x_pad = jnp.pad(in_feat, ((0, N_PAD - N), (0, 0)))
    b0r = b0.reshape(1, D)
    b1r = b1.reshape(1, D)
    b2r = b2.reshape(1, D)

    cnt_src, cnt_dst = _deg_kernel(srcp, dstp)

    h_lo, h_hi = _mm0(x_pad, cnt_src, W0)
    a_lo, a_hi = _agg_kernel(h_lo, h_hi, srcp, dstp)
    h_lo, h_hi = _mid(a_lo, a_hi, cnt_dst, cnt_src, b0r, W1)
    a_lo, a_hi = _agg_kernel(h_lo, h_hi, srcp, dstp)
    h_lo, h_hi = _mid(a_lo, a_hi, cnt_dst, cnt_src, b1r, W2)
    a_lo, a_hi = _agg_kernel(h_lo, h_hi, srcp, dstp)
    return _fin(a_lo, a_hi, cnt_dst, b2r)
